# baseline (speedup 1.0000x reference)
import functools

import jax
import jax.numpy as jnp
from jax import lax
from jax.experimental import pallas as pl
from jax.experimental.pallas import tpu as pltpu
from jax.experimental.pallas import tpu_sc as plsc

N_CORES = 2
N_SUBCORES = 16
NW = N_CORES * N_SUBCORES
CHUNK = 128
DEG_W = 16


def _make_scatter(npad, n_chunks, width, gather, fuse_pre=False):
    rows_per_tile = npad // N_SUBCORES
    n_slabs = rows_per_tile // CHUNK
    mesh = plsc.VectorSubcoreMesh(core_axis_name="c", subcore_axis_name="s")

    n_staged = n_chunks + 1
    scratch = [
        pltpu.VMEM((n_staged, CHUNK), jnp.int32),
        pltpu.VMEM((CHUNK, width), jnp.float32),
        pltpu.VMEM((CHUNK, width), jnp.float32),
        pltpu.VMEM_SHARED((npad, width), jnp.float32),
        pltpu.SemaphoreType.DMA,
        pltpu.SemaphoreType.DMA,
    ]
    if gather:
        scratch.insert(0, pltpu.VMEM((n_staged, CHUNK), jnp.int32))
        scratch.insert(3, pltpu.VMEM_SHARED((npad, width), jnp.float32))
    if fuse_pre:
        scratch += [
            pltpu.VMEM((CHUNK, width), jnp.float32),
            pltpu.VMEM((CHUNK, width), jnp.float32),
            pltpu.VMEM((1, width), jnp.float32),
        ]

    def body(*refs):
        if fuse_pre:
            (pa_hbm, pb_hbm, gg_hbm, dinv_hbm, bias_hbm,
             src_hbm, dst_hbm, out_hbm, src_v, dst_v, rows_a, gsh, zbuf,
             acc, sem, sem_b, bufc, bufd, bias_v) = refs
        elif gather:
            (g_hbm, src_hbm, dst_hbm, out_hbm,
             src_v, dst_v, rows_a, gsh, zbuf, acc, sem, sem_b) = refs
        else:
            dst_hbm, out_hbm, dst_v, rows_a, zbuf, acc, sem, sem_b = refs
        rows_v, rows_b = rows_a, zbuf
        c = lax.axis_index("c")
        s = lax.axis_index("s")
        wid = c * N_SUBCORES + s

        def zrow(i, _):
            for j in range(width // 16):
                zbuf[i, pl.ds(j * 16, 16)] = jnp.zeros((16,), jnp.float32)
                if not gather:
                    rows_v[i, pl.ds(j * 16, 16)] = jnp.ones((16,), jnp.float32)
            return 0
        lax.fori_loop(0, CHUNK, zrow, 0)

        for k in range(n_slabs):
            base = s * rows_per_tile + k * CHUNK
            pltpu.sync_copy(zbuf, acc.at[pl.ds(base, CHUNK)])

        if fuse_pre:
            pltpu.sync_copy(bias_hbm, bias_v)
        for k in range(n_slabs):
            ds = pl.ds(s * rows_per_tile + k * CHUNK, CHUNK)
            if fuse_pre:
                pltpu.sync_copy(pa_hbm.at[ds], rows_a)
                pltpu.sync_copy(pb_hbm.at[ds], zbuf)
                pltpu.sync_copy(gg_hbm.at[ds], bufc)
                pltpu.sync_copy(dinv_hbm.at[ds], bufd)

                def stage_row(r, _):
                    for j in range(width // 16):
                        sl = pl.ds(j * 16, 16)
                        d = bufd[r, sl]
                        v = rows_a[r, sl] + zbuf[r, sl] + bufc[r, sl]
                        rows_a[r, sl] = d * (d * v + bias_v[0, sl])
                    return 0
                lax.fori_loop(0, CHUNK, stage_row, 0)
                pltpu.sync_copy(rows_a, gsh.at[ds])
            elif gather:
                pltpu.sync_copy(g_hbm.at[ds], rows_a)
                pltpu.sync_copy(rows_a, gsh.at[ds])

        pltpu.sync_copy(dst_hbm.at[wid], dst_v)
        if gather:
            pltpu.sync_copy(src_hbm.at[wid], src_v)
        plsc.subcore_barrier()

        if gather:
            def edge_pair(i, _):
                jj = 2 * i
                cp_a = pltpu.async_copy(gsh.at[src_v.at[jj]], rows_a, sem)
                cp_b = pltpu.async_copy(gsh.at[src_v.at[jj + 1]], rows_b,
                                        sem_b)
                cp_a.wait()
                pltpu.sync_copy(rows_a, acc.at[dst_v.at[jj]], add=True)
                cp_b.wait()
                pltpu.sync_copy(rows_b, acc.at[dst_v.at[jj + 1]], add=True)
                return 0
            lax.fori_loop(0, n_chunks // 2, edge_pair, 0)
        else:
            def edge_chunk(j, _):
                pltpu.sync_copy(rows_v, acc.at[dst_v.at[j]], add=True)
                return 0
            lax.fori_loop(0, n_chunks, edge_chunk, 0)
        plsc.subcore_barrier()

        for k in range(n_slabs):
            base = s * rows_per_tile + k * CHUNK
            pltpu.sync_copy(acc.at[pl.ds(base, CHUNK)], zbuf)
            pltpu.sync_copy(zbuf, out_hbm.at[c, pl.ds(base, CHUNK)])

    return pl.kernel(
        body,
        out_type=jax.ShapeDtypeStruct((N_CORES, npad, width), jnp.float32),
        mesh=mesh,
        scratch_types=scratch,
        compiler_params=pltpu.CompilerParams(use_tc_tiling_on_sc=False),
    )


def _tc_call(body, n_out, npad, width):
    outs = [jax.ShapeDtypeStruct((npad, w), jnp.float32)
            for w in ([width] if n_out == 1 else width)]
    return pl.pallas_call(body, out_shape=outs[0] if n_out == 1 else outs)


def _k1_body(x_ref, dega_ref, degb_ref, w1_ref, g1_ref, dinv_ref, dinv32_ref):
    deg = dega_ref[:, 0:1] + degb_ref[:, 0:1] + 1.0
    dinv = lax.rsqrt(deg)
    g1_ref[...] = dinv * jnp.dot(x_ref[...], w1_ref[...],
                                 preferred_element_type=jnp.float32)
    dinv_ref[...] = dinv
    dinv32_ref[...] = jnp.broadcast_to(dinv, (dinv.shape[0], 32))


def _k2_body(pa, pb, g1, dinv, b1, w2, g2o):
    s1 = dinv[...] * (pa[...] + pb[...] + g1[...])
    h1 = jnp.maximum(s1 + b1[...], 0.0)
    g2o[...] = dinv[...] * jnp.dot(h1, w2[...],
                                   preferred_element_type=jnp.float32)


def _k4_body(p3a, p3b, p2a, p2b, g2, dinv, w3, b2, b3, g4o):
    g3 = dinv[...] * (dinv[...] * (p2a[...] + p2b[...] + g2[...]) + b2[...])
    s3 = dinv[...] * (p3a[...] + p3b[...] + g3)
    h3 = jnp.maximum(jnp.dot(s3, w3[...], preferred_element_type=jnp.float32)
                     + b3[...], 0.0)
    g4o[...] = dinv[...] * h3


def _k5_body(pa, pb, g4, dinv, w4, b4, out):
    s4 = dinv[...] * (pa[...] + pb[...] + g4[...])
    out[...] = jnp.dot(s4, w4[...], preferred_element_type=jnp.float32) \
        + b4[...]


def kernel(x, edge_index, W1, b1, W2, b2, W3, b3, W4, b4):
    n = x.shape[0]
    e = edge_index.shape[1]
    npad = 2048 * (-(-(n + 1) // 2048))
    dummy = n
    n_chunks = 2 * (-(-e // (NW * CHUNK * 2)))
    epad = NW * CHUNK * n_chunks

    ei = edge_index.astype(jnp.int32)
    pad_chunk = jnp.full((NW, 1, CHUNK), dummy, jnp.int32)
    src = jnp.full((epad,), dummy, jnp.int32).at[:e].set(ei[0])
    dst = jnp.full((epad,), dummy, jnp.int32).at[:e].set(ei[1])
    src3 = jnp.concatenate([src.reshape(NW, n_chunks, CHUNK), pad_chunk], 1)
    dst3 = jnp.concatenate([dst.reshape(NW, n_chunks, CHUNK), pad_chunk], 1)
    xp = jnp.zeros((npad, x.shape[1]), jnp.float32).at[:n].set(x)

    deg_scatter = _make_scatter(npad, n_chunks, DEG_W, gather=False)
    scat64 = _make_scatter(npad, n_chunks, 64, gather=True)
    scat32 = _make_scatter(npad, n_chunks, 32, gather=True)
    scat32f = _make_scatter(npad, n_chunks, 32, gather=True, fuse_pre=True)

    degp = deg_scatter(dst3)

    sds = jax.ShapeDtypeStruct
    g1, dinv, dinv32 = pl.pallas_call(
        _k1_body,
        out_shape=[sds((npad, 64), jnp.float32), sds((npad, 1), jnp.float32),
                   sds((npad, 32), jnp.float32)],
    )(xp, degp[0], degp[1], W1)

    p1 = scat64(g1, src3, dst3)
    g2 = pl.pallas_call(_k2_body, out_shape=sds((npad, 32), jnp.float32))(
        p1[0], p1[1], g1, dinv, b1.reshape(1, -1), W2)

    p2 = scat32(g2, src3, dst3)
    p3 = scat32f(p2[0], p2[1], g2, dinv32, b2.reshape(1, -1), src3, dst3)
    g4 = pl.pallas_call(_k4_body, out_shape=sds((npad, 64), jnp.float32))(
        p3[0], p3[1], p2[0], p2[1], g2, dinv, W3,
        b2.reshape(1, -1), b3.reshape(1, -1))

    p4 = scat64(g4, src3, dst3)
    out = pl.pallas_call(_k5_body, out_shape=sds((npad, 128), jnp.float32))(
        p4[0], p4[1], g4, dinv, W4, b4.reshape(1, -1))

    return out[:n]


# speedup vs baseline: 1.0395x; 1.0395x over previous
import functools

import jax
import jax.numpy as jnp
from jax import lax
from jax.experimental import pallas as pl
from jax.experimental.pallas import tpu as pltpu
from jax.experimental.pallas import tpu_sc as plsc

N_CORES = 2
N_SUBCORES = 16
NW = N_CORES * N_SUBCORES
CHUNK = 128
DEG_W = 16


def _make_scatter(npad, n_chunks, width, gather):
    rows_per_tile = npad // N_SUBCORES
    n_slabs = rows_per_tile // CHUNK
    mesh = plsc.VectorSubcoreMesh(core_axis_name="c", subcore_axis_name="s")

    n_staged = n_chunks + 1
    scratch = [
        pltpu.VMEM((n_staged, CHUNK), jnp.int32),
        pltpu.VMEM((CHUNK, width), jnp.float32),
        pltpu.VMEM((CHUNK, width), jnp.float32),
        pltpu.VMEM_SHARED((npad, width), jnp.float32),
        pltpu.SemaphoreType.DMA,
        pltpu.SemaphoreType.DMA,
    ]
    if gather:
        scratch.insert(0, pltpu.VMEM((n_staged, CHUNK), jnp.int32))
        scratch.insert(3, pltpu.VMEM_SHARED((npad, width), jnp.float32))

    def body(*refs):
        if gather:
            (g_hbm, src_hbm, dst_hbm, out_hbm,
             src_v, dst_v, rows_a, gsh, zbuf, acc, sem, sem_b) = refs
        else:
            dst_hbm, out_hbm, dst_v, rows_a, zbuf, acc, sem, sem_b = refs
        rows_v, rows_b = rows_a, zbuf
        c = lax.axis_index("c")
        s = lax.axis_index("s")
        wid = c * N_SUBCORES + s

        def zrow(i, _):
            for j in range(width // 16):
                zbuf[i, pl.ds(j * 16, 16)] = jnp.zeros((16,), jnp.float32)
                if not gather:
                    rows_v[i, pl.ds(j * 16, 16)] = jnp.ones((16,), jnp.float32)
            return 0
        lax.fori_loop(0, CHUNK, zrow, 0)

        for k in range(n_slabs):
            ds = pl.ds(s * rows_per_tile + k * CHUNK, CHUNK)
            pltpu.sync_copy(zbuf, acc.at[ds])
            if gather:
                pltpu.sync_copy(g_hbm.at[ds], gsh.at[ds])

        pltpu.sync_copy(dst_hbm.at[wid], dst_v)
        if gather:
            pltpu.sync_copy(src_hbm.at[wid], src_v)
        plsc.subcore_barrier()

        if gather:
            def edge_pair(i, _):
                jj = 2 * i
                cp_a = pltpu.async_copy(gsh.at[src_v.at[jj]], rows_a, sem)
                cp_b = pltpu.async_copy(gsh.at[src_v.at[jj + 1]], rows_b,
                                        sem_b)
                cp_a.wait()
                pltpu.sync_copy(rows_a, acc.at[dst_v.at[jj]], add=True)
                cp_b.wait()
                pltpu.sync_copy(rows_b, acc.at[dst_v.at[jj + 1]], add=True)
                return 0
            lax.fori_loop(0, n_chunks // 2, edge_pair, 0)
        else:
            def edge_chunk(j, _):
                pltpu.sync_copy(rows_v, acc.at[dst_v.at[j]], add=True)
                return 0
            lax.fori_loop(0, n_chunks, edge_chunk, 0)
        plsc.subcore_barrier()

        for k in range(n_slabs):
            base = s * rows_per_tile + k * CHUNK
            pltpu.sync_copy(acc.at[pl.ds(base, CHUNK)], zbuf)
            pltpu.sync_copy(zbuf, out_hbm.at[c, pl.ds(base, CHUNK)])

    return pl.kernel(
        body,
        out_type=jax.ShapeDtypeStruct((N_CORES, npad, width), jnp.float32),
        mesh=mesh,
        scratch_types=scratch,
        compiler_params=pltpu.CompilerParams(use_tc_tiling_on_sc=False),
    )


def _tc_call(body, n_out, npad, width):
    outs = [jax.ShapeDtypeStruct((npad, w), jnp.float32)
            for w in ([width] if n_out == 1 else width)]
    return pl.pallas_call(body, out_shape=outs[0] if n_out == 1 else outs)


def _k1_body(x_ref, dega_ref, degb_ref, w1_ref, g1_ref, dinv_ref):
    deg = dega_ref[:, 0:1] + degb_ref[:, 0:1] + 1.0
    dinv = lax.rsqrt(deg)
    g1_ref[...] = dinv * jnp.dot(x_ref[...], w1_ref[...],
                                 preferred_element_type=jnp.float32)
    dinv_ref[...] = dinv


def _k2_body(pa, pb, g1, dinv, b1, w2, g2o):
    s1 = dinv[...] * (pa[...] + pb[...] + g1[...])
    h1 = jnp.maximum(s1 + b1[...], 0.0)
    g2o[...] = dinv[...] * jnp.dot(h1, w2[...],
                                   preferred_element_type=jnp.float32)


def _k3_body(pa, pb, g2, dinv, b2, g3o):
    s2 = dinv[...] * (pa[...] + pb[...] + g2[...])
    g3o[...] = dinv[...] * (s2 + b2[...])


def _k4_body(pa, pb, g3, dinv, w3, b3, g4o):
    s3 = dinv[...] * (pa[...] + pb[...] + g3[...])
    h3 = jnp.maximum(jnp.dot(s3, w3[...], preferred_element_type=jnp.float32)
                     + b3[...], 0.0)
    g4o[...] = dinv[...] * h3


def _k5_body(pa, pb, g4, dinv, w4, b4, out):
    s4 = dinv[...] * (pa[...] + pb[...] + g4[...])
    out[...] = jnp.dot(s4, w4[...], preferred_element_type=jnp.float32) \
        + b4[...]


def kernel(x, edge_index, W1, b1, W2, b2, W3, b3, W4, b4):
    n = x.shape[0]
    e = edge_index.shape[1]
    npad = 2048 * (-(-(n + 1) // 2048))
    dummy = n
    n_chunks = 2 * (-(-e // (NW * CHUNK * 2)))
    epad = NW * CHUNK * n_chunks

    ei = edge_index.astype(jnp.int32)
    pad_chunk = jnp.full((NW, 1, CHUNK), dummy, jnp.int32)
    src = jnp.full((epad,), dummy, jnp.int32).at[:e].set(ei[0])
    dst = jnp.full((epad,), dummy, jnp.int32).at[:e].set(ei[1])
    src3 = jnp.concatenate([src.reshape(NW, n_chunks, CHUNK), pad_chunk], 1)
    dst3 = jnp.concatenate([dst.reshape(NW, n_chunks, CHUNK), pad_chunk], 1)
    xp = jnp.zeros((npad, x.shape[1]), jnp.float32).at[:n].set(x)

    deg_scatter = _make_scatter(npad, n_chunks, DEG_W, gather=False)
    scat64 = _make_scatter(npad, n_chunks, 64, gather=True)
    scat32 = _make_scatter(npad, n_chunks, 32, gather=True)

    degp = deg_scatter(dst3)

    sds = jax.ShapeDtypeStruct
    g1, dinv = pl.pallas_call(
        _k1_body,
        out_shape=[sds((npad, 64), jnp.float32), sds((npad, 1), jnp.float32)],
    )(xp, degp[0], degp[1], W1)

    p1 = scat64(g1, src3, dst3)
    g2 = pl.pallas_call(_k2_body, out_shape=sds((npad, 32), jnp.float32))(
        p1[0], p1[1], g1, dinv, b1.reshape(1, -1), W2)

    p2 = scat32(g2, src3, dst3)
    g3 = pl.pallas_call(_k3_body, out_shape=sds((npad, 32), jnp.float32))(
        p2[0], p2[1], g2, dinv, b2.reshape(1, -1))

    p3 = scat32(g3, src3, dst3)
    g4 = pl.pallas_call(_k4_body, out_shape=sds((npad, 64), jnp.float32))(
        p3[0], p3[1], g3, dinv, W3, b3.reshape(1, -1))

    p4 = scat64(g4, src3, dst3)
    out = pl.pallas_call(_k5_body, out_shape=sds((npad, 128), jnp.float32))(
        p4[0], p4[1], g4, dinv, W4, b4.reshape(1, -1))

    return out[:n]
